# baseline (speedup 1.0000x reference)
import functools

import jax
import jax.numpy as jnp
from jax.experimental import pallas as pl
from jax.experimental.pallas import tpu as pltpu


def _se_gate_kernel(x_ref, w1_ref, w2_ref, g_ref, *, inv_s):
    se = jnp.sum(x_ref[...].astype(jnp.float32), axis=-1) * inv_s
    h = jnp.maximum(
        jnp.dot(se, w1_ref[...].astype(jnp.float32),
                preferred_element_type=jnp.float32), 0.0)
    g = jax.nn.sigmoid(
        jnp.dot(h, w2_ref[...].astype(jnp.float32),
                preferred_element_type=jnp.float32))
    g_ref[...] = g[:, None, :]


def kernel(x, w1, w2):
    B, C, D, H, W = x.shape
    Cr = w1.shape[1]
    S = D * H * W
    xf = x.reshape(B, C, S).astype(jnp.bfloat16)

    TB = 2 if B % 2 == 0 else 1
    grid = (B // TB,)

    g = pl.pallas_call(
        functools.partial(_se_gate_kernel, inv_s=1.0 / float(S)),
        out_shape=jax.ShapeDtypeStruct((B, 1, C), jnp.float32),
        grid=grid,
        in_specs=[
            pl.BlockSpec((TB, C, S), lambda b: (b, 0, 0)),
            pl.BlockSpec((C, Cr), lambda b: (0, 0)),
            pl.BlockSpec((Cr, C), lambda b: (0, 0)),
        ],
        out_specs=pl.BlockSpec((TB, 1, C), lambda b: (b, 0, 0)),
        compiler_params=pltpu.CompilerParams(
            dimension_semantics=("arbitrary",),
            vmem_limit_bytes=56 * 1024 * 1024),
    )(xf, w1, w2)

    gate = g.reshape(B, C).astype(x.dtype)
    return x * gate[:, :, None, None, None]


# speedup vs baseline: 1.0554x; 1.0554x over previous
import functools

import jax
import jax.numpy as jnp
from jax.experimental import pallas as pl
from jax.experimental.pallas import tpu as pltpu


def _se_gate_kernel(x_ref, w1_ref, w2_ref, g_ref, *, inv_s):
    se = jnp.sum(x_ref[...].astype(jnp.float32), axis=-1) * inv_s
    h = jnp.maximum(
        jnp.dot(se, w1_ref[...].astype(jnp.float32),
                preferred_element_type=jnp.float32), 0.0)
    g = jax.nn.sigmoid(
        jnp.dot(h, w2_ref[...].astype(jnp.float32),
                preferred_element_type=jnp.float32))
    g_ref[...] = g[:, None, :]


def kernel(x, w1, w2):
    B, C, D, H, W = x.shape
    Cr = w1.shape[1]
    S = D * H * W
    xf = x.reshape(B, C, S)

    TB = 4 if B % 4 == 0 else 1
    grid = (B // TB,)

    g = pl.pallas_call(
        functools.partial(_se_gate_kernel, inv_s=1.0 / float(S)),
        out_shape=jax.ShapeDtypeStruct((B, 1, C), jnp.float32),
        grid=grid,
        in_specs=[
            pl.BlockSpec((TB, C, S), lambda b: (b, 0, 0)),
            pl.BlockSpec((C, Cr), lambda b: (0, 0)),
            pl.BlockSpec((Cr, C), lambda b: (0, 0)),
        ],
        out_specs=pl.BlockSpec((TB, 1, C), lambda b: (b, 0, 0)),
        compiler_params=pltpu.CompilerParams(
            dimension_semantics=("arbitrary",),
            vmem_limit_bytes=56 * 1024 * 1024),
    )(xf, w1, w2)

    gate = g.reshape(B, C).astype(x.dtype)
    return x * gate[:, :, None, None, None]
